# baseline (speedup 1.0000x reference)
import functools

import jax
import jax.numpy as jnp
from jax import lax
from jax.experimental import pallas as pl
from jax.experimental.pallas import tpu as pltpu
from jax.experimental.pallas import tpu_sc as plsc

N = 10000
NP = 10240
NACC = 10240
E_RAW = 320000
NW = 32
K = 80
EPT = E_RAW // NW
NCHUNK = EPT // K
ROWS_PER_SUB = NACC // 16

BLK = 512
GRID = NP // BLK


def _dyn_gather(v, idx):
    return lax.gather(
        v, idx[:, None],
        lax.GatherDimensionNumbers(
            offset_dims=(), collapsed_slice_dims=(0,), start_index_map=(0,)),
        slice_sizes=(1,),
        mode=lax.GatherScatterMode.PROMISE_IN_BOUNDS)


def _dot(a, b, prec):
    return lax.dot_general(a, b, (((1,), (0,)), ((), ())),
                           preferred_element_type=jnp.float32,
                           precision=prec)


_DEF = lax.Precision.DEFAULT
_HI = lax.Precision.HIGHEST


def _node1_body(x_ref, w1_ref, as_ref, ad_ref, tab_ref, adout_ref):
    h = _dot(x_ref[...], w1_ref[...], _DEF)
    a_s = _dot(h, as_ref[...], _HI)
    a_d = _dot(h, ad_ref[...], _HI)
    z = jnp.zeros((h.shape[0], 8), jnp.float32)
    tab_ref[...] = jnp.concatenate([h, a_s, z], axis=1)
    adout_ref[...] = jnp.concatenate([a_d, z], axis=1)


def _norm1_body(a0_ref, a1_ref, tab_ref, ad_ref, e8_ref, b1_ref, w2_ref,
                p2_ref, tab2_ref):
    s = a0_ref[...] + a1_ref[...]
    tab = tab_ref[...]
    h = tab[:, 0:64]
    t = tab[:, 64:72] + ad_ref[:, 0:8]
    t = jnp.where(t >= 0, t, 0.2 * t)
    wself = jnp.exp(t)
    den = s[:, 64:72] + wself
    wself_b = _dot(wself, e8_ref[...], _HI)
    num = s[:, 0:64] + wself_b * h
    den_b = _dot(den, e8_ref[...], _HI)
    h1 = num / den_b + b1_ref[0:1, :]
    h1 = jnp.where(h1 > 0, h1, jnp.exp(h1) - 1.0)
    h2 = _dot(h1, w2_ref[...], _DEF)
    tab2_ref[...] = _dot(h2, p2_ref[...], _HI)


def _final_body(a0_ref, a1_ref, tab2_ref, sel3_ref, sel03_ref, b2e_ref,
                wl16_ref, c0_ref, out_ref):
    s = a0_ref[...] + a1_ref[...]
    tab2 = tab2_ref[...]
    t = tab2[:, 3:4] + tab2[:, 4:5]
    t = jnp.where(t >= 0, t, 0.2 * t)
    wself = jnp.exp(t)
    den = s[:, 0:1] + wself
    num_sel = _dot(s, sel3_ref[...], _HI)
    self_sel = _dot(tab2, sel03_ref[...], _HI)
    h2e = (num_sel + wself * self_sel) / den + b2e_ref[0:1, :]
    out_ref[...] = _dot(h2e, wl16_ref[...], _DEF) + c0_ref[0:1, :]


_MESH = plsc.VectorSubcoreMesh(core_axis_name="c", subcore_axis_name="s")


def _edge1_body(src_hbm, dst_hbm, tab_hbm, ad_hbm, out0_hbm, out1_hbm,
                sidx_all, didx_all, gbuf0, adbuf0, obuf0,
                gbuf1, adbuf1, obuf1, acc, gsem0, gsem1, ssem0, ssem1):
    c = lax.axis_index("c")
    s = lax.axis_index("s")
    wid = s * 2 + c

    pltpu.sync_copy(src_hbm.at[wid], sidx_all)
    pltpu.sync_copy(dst_hbm.at[wid], didx_all)

    def _zrow(r, _):
        for j in range(5):
            obuf0[r, pl.ds(16 * j, 16)] = jnp.zeros((16,), jnp.float32)
            obuf1[r, pl.ds(16 * j, 16)] = jnp.zeros((16,), jnp.float32)
        return 0
    lax.fori_loop(0, K, _zrow, 0)
    for t in range(ROWS_PER_SUB // K):
        pltpu.sync_copy(obuf0, acc.at[pl.ds(s * ROWS_PER_SUB + t * K, K)])
    plsc.subcore_barrier()

    pltpu.async_copy(obuf0, acc.at[didx_all.at[0]], ssem0, add=True)
    pltpu.async_copy(obuf1, acc.at[didx_all.at[0]], ssem1, add=True)

    lane = lax.iota(jnp.int32, 16)
    hi = jnp.where(lane >= 8, 1, 0)

    def _issue(j, gb, ab, sem):
        pltpu.async_copy(tab_hbm.at[sidx_all.at[j]], gb, sem)
        pltpu.async_copy(ad_hbm.at[didx_all.at[j]], ab, sem)

    def _wait_g(gb, ab, sem):
        pltpu.make_async_copy(tab_hbm.at[sidx_all.at[0]], gb, sem).wait()
        pltpu.make_async_copy(ad_hbm.at[didx_all.at[0]], ab, sem).wait()

    def _wait_s(ob, sem):
        pltpu.make_async_copy(ob, acc.at[didx_all.at[0]], sem).wait()

    def _compute(gb, ab, ob):
        @plsc.parallel_loop(0, K, unroll=8)
        def _edge(e):
            va = gb[e, pl.ds(64, 16)]
            vd = ab[e, :]
            t = va + vd
            t = jnp.where(t >= 0, t, 0.2 * t)
            w = jnp.exp(t)
            for j in range(4):
                wp = _dyn_gather(w, hi + 2 * j)
                ob[e, pl.ds(16 * j, 16)] = gb[e, pl.ds(16 * j, 16)] * wp
            ob[e, pl.ds(64, 16)] = w

    _issue(0, gbuf0, adbuf0, gsem0)

    def _outer(g, _):
        k0 = 2 * g
        _issue(k0 + 1, gbuf1, adbuf1, gsem1)
        _wait_g(gbuf0, adbuf0, gsem0)
        _wait_s(obuf0, ssem0)
        _compute(gbuf0, adbuf0, obuf0)
        pltpu.async_copy(obuf0, acc.at[didx_all.at[k0]], ssem0, add=True)
        _issue(k0 + 2, gbuf0, adbuf0, gsem0)
        _wait_g(gbuf1, adbuf1, gsem1)
        _wait_s(obuf1, ssem1)
        _compute(gbuf1, adbuf1, obuf1)
        pltpu.async_copy(obuf1, acc.at[didx_all.at[k0 + 1]], ssem1, add=True)
        return 0
    lax.fori_loop(0, NCHUNK // 2, _outer, 0)
    _wait_g(gbuf0, adbuf0, gsem0)
    _wait_s(obuf0, ssem0)
    _compute(gbuf0, adbuf0, obuf0)
    pltpu.async_copy(obuf0, acc.at[didx_all.at[NCHUNK - 1]], ssem0, add=True)
    _wait_s(obuf0, ssem0)
    _wait_s(obuf1, ssem1)

    plsc.subcore_barrier()
    rows = acc.at[pl.ds(s * ROWS_PER_SUB, ROWS_PER_SUB)]

    @pl.when(c == 0)
    def _():
        pltpu.sync_copy(rows, out0_hbm.at[pl.ds(s * ROWS_PER_SUB,
                                                ROWS_PER_SUB)])

    @pl.when(c == 1)
    def _():
        pltpu.sync_copy(rows, out1_hbm.at[pl.ds(s * ROWS_PER_SUB,
                                                ROWS_PER_SUB)])


def _edge2_body(src_hbm, dst_hbm, tab_hbm, out0_hbm, out1_hbm,
                sidx_all, didx_all, sbuf0, dbuf0, obuf0,
                sbuf1, dbuf1, obuf1, acc, gsem0, gsem1, ssem0, ssem1):
    c = lax.axis_index("c")
    s = lax.axis_index("s")
    wid = s * 2 + c

    pltpu.sync_copy(src_hbm.at[wid], sidx_all)
    pltpu.sync_copy(dst_hbm.at[wid], didx_all)

    def _zrow(r, _):
        obuf0[r, :] = jnp.zeros((16,), jnp.float32)
        obuf1[r, :] = jnp.zeros((16,), jnp.float32)
        return 0
    lax.fori_loop(0, K, _zrow, 0)
    for t in range(ROWS_PER_SUB // K):
        pltpu.sync_copy(obuf0, acc.at[pl.ds(s * ROWS_PER_SUB + t * K, K)])
    plsc.subcore_barrier()

    pltpu.async_copy(obuf0, acc.at[didx_all.at[0]], ssem0, add=True)
    pltpu.async_copy(obuf1, acc.at[didx_all.at[0]], ssem1, add=True)

    lane = lax.iota(jnp.int32, 16)
    three = lane * 0 + 3
    four = lane * 0 + 4
    sel_idx = jnp.where(lane <= 3, jnp.maximum(lane - 1, 0), 5)

    def _issue(j, sb, db, sem):
        pltpu.async_copy(tab_hbm.at[sidx_all.at[j]], sb, sem)
        pltpu.async_copy(tab_hbm.at[didx_all.at[j]], db, sem)

    def _wait_g(sb, db, sem):
        pltpu.make_async_copy(tab_hbm.at[sidx_all.at[0]], sb, sem).wait()
        pltpu.make_async_copy(tab_hbm.at[didx_all.at[0]], db, sem).wait()

    def _wait_s(ob, sem):
        pltpu.make_async_copy(ob, acc.at[didx_all.at[0]], sem).wait()

    def _compute(sb, db, ob):
        @plsc.parallel_loop(0, K, unroll=8)
        def _edge(e):
            va = sb[e, :]
            vd = db[e, :]
            t = _dyn_gather(va, three) + _dyn_gather(vd, four)
            t = jnp.where(t >= 0, t, 0.2 * t)
            w = jnp.exp(t)
            shifted = _dyn_gather(va, sel_idx)
            sel = jnp.where(lane == 0, 1.0, shifted)
            ob[e, :] = w * sel

    _issue(0, sbuf0, dbuf0, gsem0)

    def _outer(g, _):
        k0 = 2 * g
        _issue(k0 + 1, sbuf1, dbuf1, gsem1)
        _wait_g(sbuf0, dbuf0, gsem0)
        _wait_s(obuf0, ssem0)
        _compute(sbuf0, dbuf0, obuf0)
        pltpu.async_copy(obuf0, acc.at[didx_all.at[k0]], ssem0, add=True)
        _issue(k0 + 2, sbuf0, dbuf0, gsem0)
        _wait_g(sbuf1, dbuf1, gsem1)
        _wait_s(obuf1, ssem1)
        _compute(sbuf1, dbuf1, obuf1)
        pltpu.async_copy(obuf1, acc.at[didx_all.at[k0 + 1]], ssem1, add=True)
        return 0
    lax.fori_loop(0, NCHUNK // 2, _outer, 0)
    _wait_g(sbuf0, dbuf0, gsem0)
    _wait_s(obuf0, ssem0)
    _compute(sbuf0, dbuf0, obuf0)
    pltpu.async_copy(obuf0, acc.at[didx_all.at[NCHUNK - 1]], ssem0, add=True)
    _wait_s(obuf0, ssem0)
    _wait_s(obuf1, ssem1)

    plsc.subcore_barrier()
    rows = acc.at[pl.ds(s * ROWS_PER_SUB, ROWS_PER_SUB)]

    @pl.when(c == 0)
    def _():
        pltpu.sync_copy(rows, out0_hbm.at[pl.ds(s * ROWS_PER_SUB,
                                                ROWS_PER_SUB)])

    @pl.when(c == 1)
    def _():
        pltpu.sync_copy(rows, out1_hbm.at[pl.ds(s * ROWS_PER_SUB,
                                                ROWS_PER_SUB)])


_SC_PARAMS = pltpu.CompilerParams(use_tc_tiling_on_sc=False)

_edge1 = pl.kernel(
    _edge1_body,
    out_type=(jax.ShapeDtypeStruct((NACC, 80), jnp.float32),
              jax.ShapeDtypeStruct((NACC, 80), jnp.float32)),
    mesh=_MESH,
    compiler_params=_SC_PARAMS,
    scratch_types=[
        pltpu.VMEM((NCHUNK, K), jnp.int32),
        pltpu.VMEM((NCHUNK, K), jnp.int32),
        pltpu.VMEM((K, 80), jnp.float32),
        pltpu.VMEM((K, 16), jnp.float32),
        pltpu.VMEM((K, 80), jnp.float32),
        pltpu.VMEM((K, 80), jnp.float32),
        pltpu.VMEM((K, 16), jnp.float32),
        pltpu.VMEM((K, 80), jnp.float32),
        pltpu.VMEM_SHARED((NACC, 80), jnp.float32),
        pltpu.SemaphoreType.DMA,
        pltpu.SemaphoreType.DMA,
        pltpu.SemaphoreType.DMA,
        pltpu.SemaphoreType.DMA,
    ])

_edge2 = pl.kernel(
    _edge2_body,
    out_type=(jax.ShapeDtypeStruct((NACC, 16), jnp.float32),
              jax.ShapeDtypeStruct((NACC, 16), jnp.float32)),
    mesh=_MESH,
    compiler_params=_SC_PARAMS,
    scratch_types=[
        pltpu.VMEM((NCHUNK, K), jnp.int32),
        pltpu.VMEM((NCHUNK, K), jnp.int32),
        pltpu.VMEM((K, 16), jnp.float32),
        pltpu.VMEM((K, 16), jnp.float32),
        pltpu.VMEM((K, 16), jnp.float32),
        pltpu.VMEM((K, 16), jnp.float32),
        pltpu.VMEM((K, 16), jnp.float32),
        pltpu.VMEM((K, 16), jnp.float32),
        pltpu.VMEM_SHARED((NACC, 16), jnp.float32),
        pltpu.SemaphoreType.DMA,
        pltpu.SemaphoreType.DMA,
        pltpu.SemaphoreType.DMA,
        pltpu.SemaphoreType.DMA,
    ])


@jax.jit
def kernel(x, edge_index, W1, att_src1, att_dst1, b1,
           W2, att_src2, att_dst2, b2, Wl, bl):
    f32 = jnp.float32
    ei = edge_index.astype(jnp.int32)
    src = ei[0].reshape(NW, NCHUNK, K)
    dst = ei[1].reshape(NW, NCHUNK, K)
    x_pad = jnp.pad(x, ((0, NP - N), (0, 0)))

    H1, C1 = att_src1.shape[1], att_src1.shape[2]
    eyeH = jnp.eye(H1, dtype=f32)
    As = (eyeH[:, None, :] * att_src1[0][:, :, None]).reshape(H1 * C1, H1)
    Ad = (eyeH[:, None, :] * att_dst1[0][:, :, None]).reshape(H1 * C1, H1)
    e8 = jnp.kron(jnp.eye(8, dtype=f32), jnp.ones((1, 8), f32))
    b1m = jnp.broadcast_to(b1[None, :], (8, 64))
    w2pad = jnp.pad(W2, ((0, 0), (0, 8 - W2.shape[1])))
    p2 = jnp.zeros((8, 16), f32)
    p2 = p2.at[jnp.arange(3), jnp.arange(3)].set(1.0)
    p2 = p2.at[0:3, 3].set(att_src2[0, 0, :])
    p2 = p2.at[0:3, 4].set(att_dst2[0, 0, :])
    sel3 = jnp.zeros((16, 16), f32).at[jnp.arange(1, 4),
                                       jnp.arange(0, 3)].set(1.0)
    sel03 = jnp.zeros((16, 16), f32).at[jnp.arange(0, 3),
                                        jnp.arange(0, 3)].set(1.0)
    b2e = jnp.zeros((8, 16), f32).at[:, 0:3].set(
        jnp.broadcast_to(b2[None, :], (8, 3)))
    wl16 = jnp.zeros((16, 8), f32).at[0:3, :].set(
        jnp.broadcast_to(Wl, (3, 8)))
    c0 = jnp.broadcast_to(bl.reshape(1, 1), (8, 8))

    tab1, ad1 = pl.pallas_call(
        _node1_body,
        grid=(GRID,),
        in_specs=[
            pl.BlockSpec((BLK, 128), lambda i: (i, 0)),
            pl.BlockSpec((128, 64), lambda i: (0, 0)),
            pl.BlockSpec((64, 8), lambda i: (0, 0)),
            pl.BlockSpec((64, 8), lambda i: (0, 0)),
        ],
        out_specs=[
            pl.BlockSpec((BLK, 80), lambda i: (i, 0)),
            pl.BlockSpec((BLK, 16), lambda i: (i, 0)),
        ],
        out_shape=[
            jax.ShapeDtypeStruct((NP, 80), f32),
            jax.ShapeDtypeStruct((NP, 16), f32),
        ],
    )(x_pad, W1, As, Ad)

    acc1a, acc1b = _edge1(src, dst, tab1, ad1)

    tab2 = pl.pallas_call(
        _norm1_body,
        grid=(GRID,),
        in_specs=[
            pl.BlockSpec((BLK, 80), lambda i: (i, 0)),
            pl.BlockSpec((BLK, 80), lambda i: (i, 0)),
            pl.BlockSpec((BLK, 80), lambda i: (i, 0)),
            pl.BlockSpec((BLK, 16), lambda i: (i, 0)),
            pl.BlockSpec((8, 64), lambda i: (0, 0)),
            pl.BlockSpec((8, 64), lambda i: (0, 0)),
            pl.BlockSpec((64, 8), lambda i: (0, 0)),
            pl.BlockSpec((8, 16), lambda i: (0, 0)),
        ],
        out_specs=pl.BlockSpec((BLK, 16), lambda i: (i, 0)),
        out_shape=jax.ShapeDtypeStruct((NP, 16), f32),
    )(acc1a, acc1b, tab1, ad1, e8, b1m, w2pad, p2)

    acc2a, acc2b = _edge2(src, dst, tab2)

    out = pl.pallas_call(
        _final_body,
        grid=(GRID,),
        in_specs=[
            pl.BlockSpec((BLK, 16), lambda i: (i, 0)),
            pl.BlockSpec((BLK, 16), lambda i: (i, 0)),
            pl.BlockSpec((BLK, 16), lambda i: (i, 0)),
            pl.BlockSpec((16, 16), lambda i: (0, 0)),
            pl.BlockSpec((16, 16), lambda i: (0, 0)),
            pl.BlockSpec((8, 16), lambda i: (0, 0)),
            pl.BlockSpec((16, 8), lambda i: (0, 0)),
            pl.BlockSpec((8, 8), lambda i: (0, 0)),
        ],
        out_specs=pl.BlockSpec((BLK, 8), lambda i: (i, 0)),
        out_shape=jax.ShapeDtypeStruct((NP, 8), f32),
    )(acc2a, acc2b, tab2, sel3, sel03, b2e, wl16, c0)

    return out[:N, 0:1]


# speedup vs baseline: 1.0013x; 1.0013x over previous
import jax
import jax.numpy as jnp
from jax import lax
from jax.experimental import pallas as pl
from jax.experimental.pallas import tpu as pltpu
from jax.experimental.pallas import tpu_sc as plsc

N = 10000
NP = 10240
NACC = 10240
E_RAW = 320000
NW = 32
K = 80
EPT = E_RAW // NW
NCHUNK = EPT // K
ROWS_PER_SUB = NACC // 16

BLK = 512
GRID = NP // BLK


def _dyn_gather(v, idx):
    return lax.gather(
        v, idx[:, None],
        lax.GatherDimensionNumbers(
            offset_dims=(), collapsed_slice_dims=(0,), start_index_map=(0,)),
        slice_sizes=(1,),
        mode=lax.GatherScatterMode.PROMISE_IN_BOUNDS)


def _dot(a, b, prec):
    return lax.dot_general(a, b, (((1,), (0,)), ((), ())),
                           preferred_element_type=jnp.float32,
                           precision=prec)


_DEF = lax.Precision.DEFAULT
_HI = lax.Precision.HIGHEST


def _node1_body(x_ref, w1_ref, as_ref, ad_ref, tab_ref, adout_ref):
    h = _dot(x_ref[...], w1_ref[...], _DEF)
    a_s = _dot(h, as_ref[...], _HI)
    a_d = _dot(h, ad_ref[...], _HI)
    z = jnp.zeros((h.shape[0], 8), jnp.float32)
    tab_ref[...] = jnp.concatenate([h, a_s, z], axis=1)
    adout_ref[...] = jnp.concatenate([a_d, z], axis=1)


def _norm1_body(a0_ref, a1_ref, tab_ref, ad_ref, e8_ref, b1_ref, w2_ref,
                p2_ref, tab2_ref):
    s = a0_ref[...] + a1_ref[...]
    tab = tab_ref[...]
    h = tab[:, 0:64]
    t = tab[:, 64:72] + ad_ref[:, 0:8]
    t = jnp.where(t >= 0, t, 0.2 * t)
    wself = jnp.exp(t)
    den = s[:, 64:72] + wself
    wself_b = _dot(wself, e8_ref[...], _HI)
    num = s[:, 0:64] + wself_b * h
    den_b = _dot(den, e8_ref[...], _HI)
    h1 = num / den_b + b1_ref[0:1, :]
    h1 = jnp.where(h1 > 0, h1, jnp.exp(h1) - 1.0)
    h2 = _dot(h1, w2_ref[...], _DEF)
    tab2_ref[...] = _dot(h2, p2_ref[...], _HI)


def _final_body(a0_ref, a1_ref, tab2_ref, sel3_ref, sel03_ref, b2e_ref,
                wl16_ref, c0_ref, out_ref):
    s = a0_ref[...] + a1_ref[...]
    tab2 = tab2_ref[...]
    t = tab2[:, 3:4] + tab2[:, 4:5]
    t = jnp.where(t >= 0, t, 0.2 * t)
    wself = jnp.exp(t)
    den = s[:, 0:1] + wself
    num_sel = _dot(s, sel3_ref[...], _HI)
    self_sel = _dot(tab2, sel03_ref[...], _HI)
    h2e = (num_sel + wself * self_sel) / den + b2e_ref[0:1, :]
    out_ref[...] = _dot(h2e, wl16_ref[...], _DEF) + c0_ref[0:1, :]


_MESH = plsc.VectorSubcoreMesh(core_axis_name="c", subcore_axis_name="s")


def _edge1_body(src_hbm, dst_hbm, tab_hbm, ad_hbm, out0_hbm, out1_hbm,
                sidx_all, didx_all, gbuf0, adbuf0, obuf0,
                gbuf1, adbuf1, obuf1, acc, gsem0, gsem1, ssem0, ssem1):
    c = lax.axis_index("c")
    s = lax.axis_index("s")
    wid = s * 2 + c

    pltpu.sync_copy(src_hbm.at[wid], sidx_all)
    pltpu.sync_copy(dst_hbm.at[wid], didx_all)

    def _zrow(r, _):
        for j in range(5):
            obuf0[r, pl.ds(16 * j, 16)] = jnp.zeros((16,), jnp.float32)
            obuf1[r, pl.ds(16 * j, 16)] = jnp.zeros((16,), jnp.float32)
        return 0
    lax.fori_loop(0, K, _zrow, 0)
    for t in range(ROWS_PER_SUB // K):
        pltpu.sync_copy(obuf0, acc.at[pl.ds(s * ROWS_PER_SUB + t * K, K)])
    plsc.subcore_barrier()

    pltpu.async_copy(obuf0, acc.at[didx_all.at[0]], ssem0, add=True)
    pltpu.async_copy(obuf1, acc.at[didx_all.at[0]], ssem1, add=True)

    lane = lax.iota(jnp.int32, 16)
    hi = jnp.where(lane >= 8, 1, 0)

    def _issue(j, gb, ab, sem):
        pltpu.async_copy(tab_hbm.at[sidx_all.at[j]], gb, sem)
        pltpu.async_copy(ad_hbm.at[didx_all.at[j]], ab, sem)

    def _wait_g(gb, ab, sem):
        pltpu.make_async_copy(tab_hbm.at[sidx_all.at[0]], gb, sem).wait()
        pltpu.make_async_copy(ad_hbm.at[didx_all.at[0]], ab, sem).wait()

    def _wait_s(ob, sem):
        pltpu.make_async_copy(ob, acc.at[didx_all.at[0]], sem).wait()

    def _compute(gb, ab, ob):
        @plsc.parallel_loop(0, K, unroll=4)
        def _edge(e):
            va = gb[e, pl.ds(64, 16)]
            vd = ab[e, :]
            t = va + vd
            t = jnp.where(t >= 0, t, 0.2 * t)
            w = jnp.exp(t)
            for j in range(4):
                wp = _dyn_gather(w, hi + 2 * j)
                ob[e, pl.ds(16 * j, 16)] = gb[e, pl.ds(16 * j, 16)] * wp
            ob[e, pl.ds(64, 16)] = w

    _issue(0, gbuf0, adbuf0, gsem0)

    def _outer(g, _):
        k0 = 2 * g
        _issue(k0 + 1, gbuf1, adbuf1, gsem1)
        _wait_g(gbuf0, adbuf0, gsem0)
        _wait_s(obuf0, ssem0)
        _compute(gbuf0, adbuf0, obuf0)
        pltpu.async_copy(obuf0, acc.at[didx_all.at[k0]], ssem0, add=True)
        _issue(k0 + 2, gbuf0, adbuf0, gsem0)
        _wait_g(gbuf1, adbuf1, gsem1)
        _wait_s(obuf1, ssem1)
        _compute(gbuf1, adbuf1, obuf1)
        pltpu.async_copy(obuf1, acc.at[didx_all.at[k0 + 1]], ssem1, add=True)
        return 0
    lax.fori_loop(0, NCHUNK // 2, _outer, 0)
    _wait_g(gbuf0, adbuf0, gsem0)
    _wait_s(obuf0, ssem0)
    _compute(gbuf0, adbuf0, obuf0)
    pltpu.async_copy(obuf0, acc.at[didx_all.at[NCHUNK - 1]], ssem0, add=True)
    _wait_s(obuf0, ssem0)
    _wait_s(obuf1, ssem1)

    plsc.subcore_barrier()
    rows = acc.at[pl.ds(s * ROWS_PER_SUB, ROWS_PER_SUB)]

    @pl.when(c == 0)
    def _():
        pltpu.sync_copy(rows, out0_hbm.at[pl.ds(s * ROWS_PER_SUB,
                                                ROWS_PER_SUB)])

    @pl.when(c == 1)
    def _():
        pltpu.sync_copy(rows, out1_hbm.at[pl.ds(s * ROWS_PER_SUB,
                                                ROWS_PER_SUB)])


def _edge2_body(src_hbm, dst_hbm, tab_hbm, out0_hbm, out1_hbm,
                sidx_all, didx_all, sbuf0, dbuf0, obuf0,
                sbuf1, dbuf1, obuf1, acc, gsem0, gsem1, ssem0, ssem1):
    c = lax.axis_index("c")
    s = lax.axis_index("s")
    wid = s * 2 + c

    pltpu.sync_copy(src_hbm.at[wid], sidx_all)
    pltpu.sync_copy(dst_hbm.at[wid], didx_all)

    def _zrow(r, _):
        obuf0[r, :] = jnp.zeros((16,), jnp.float32)
        obuf1[r, :] = jnp.zeros((16,), jnp.float32)
        return 0
    lax.fori_loop(0, K, _zrow, 0)
    for t in range(ROWS_PER_SUB // K):
        pltpu.sync_copy(obuf0, acc.at[pl.ds(s * ROWS_PER_SUB + t * K, K)])
    plsc.subcore_barrier()

    pltpu.async_copy(obuf0, acc.at[didx_all.at[0]], ssem0, add=True)
    pltpu.async_copy(obuf1, acc.at[didx_all.at[0]], ssem1, add=True)

    lane = lax.iota(jnp.int32, 16)
    three = lane * 0 + 3
    four = lane * 0 + 4
    sel_idx = jnp.where(lane <= 3, jnp.maximum(lane - 1, 0), 5)

    def _issue(j, sb, db, sem):
        pltpu.async_copy(tab_hbm.at[sidx_all.at[j]], sb, sem)
        pltpu.async_copy(tab_hbm.at[didx_all.at[j]], db, sem)

    def _wait_g(sb, db, sem):
        pltpu.make_async_copy(tab_hbm.at[sidx_all.at[0]], sb, sem).wait()
        pltpu.make_async_copy(tab_hbm.at[didx_all.at[0]], db, sem).wait()

    def _wait_s(ob, sem):
        pltpu.make_async_copy(ob, acc.at[didx_all.at[0]], sem).wait()

    def _compute(sb, db, ob):
        @plsc.parallel_loop(0, K, unroll=4)
        def _edge(e):
            va = sb[e, :]
            vd = db[e, :]
            t = _dyn_gather(va, three) + _dyn_gather(vd, four)
            t = jnp.where(t >= 0, t, 0.2 * t)
            w = jnp.exp(t)
            shifted = _dyn_gather(va, sel_idx)
            sel = jnp.where(lane == 0, 1.0, shifted)
            ob[e, :] = w * sel

    _issue(0, sbuf0, dbuf0, gsem0)

    def _outer(g, _):
        k0 = 2 * g
        _issue(k0 + 1, sbuf1, dbuf1, gsem1)
        _wait_g(sbuf0, dbuf0, gsem0)
        _wait_s(obuf0, ssem0)
        _compute(sbuf0, dbuf0, obuf0)
        pltpu.async_copy(obuf0, acc.at[didx_all.at[k0]], ssem0, add=True)
        _issue(k0 + 2, sbuf0, dbuf0, gsem0)
        _wait_g(sbuf1, dbuf1, gsem1)
        _wait_s(obuf1, ssem1)
        _compute(sbuf1, dbuf1, obuf1)
        pltpu.async_copy(obuf1, acc.at[didx_all.at[k0 + 1]], ssem1, add=True)
        return 0
    lax.fori_loop(0, NCHUNK // 2, _outer, 0)
    _wait_g(sbuf0, dbuf0, gsem0)
    _wait_s(obuf0, ssem0)
    _compute(sbuf0, dbuf0, obuf0)
    pltpu.async_copy(obuf0, acc.at[didx_all.at[NCHUNK - 1]], ssem0, add=True)
    _wait_s(obuf0, ssem0)
    _wait_s(obuf1, ssem1)

    plsc.subcore_barrier()
    rows = acc.at[pl.ds(s * ROWS_PER_SUB, ROWS_PER_SUB)]

    @pl.when(c == 0)
    def _():
        pltpu.sync_copy(rows, out0_hbm.at[pl.ds(s * ROWS_PER_SUB,
                                                ROWS_PER_SUB)])

    @pl.when(c == 1)
    def _():
        pltpu.sync_copy(rows, out1_hbm.at[pl.ds(s * ROWS_PER_SUB,
                                                ROWS_PER_SUB)])


_SC_PARAMS = pltpu.CompilerParams(use_tc_tiling_on_sc=False)

_edge1 = pl.kernel(
    _edge1_body,
    out_type=(jax.ShapeDtypeStruct((NACC, 80), jnp.float32),
              jax.ShapeDtypeStruct((NACC, 80), jnp.float32)),
    mesh=_MESH,
    compiler_params=_SC_PARAMS,
    scratch_types=[
        pltpu.VMEM((NCHUNK, K), jnp.int32),
        pltpu.VMEM((NCHUNK, K), jnp.int32),
        pltpu.VMEM((K, 80), jnp.float32),
        pltpu.VMEM((K, 16), jnp.float32),
        pltpu.VMEM((K, 80), jnp.float32),
        pltpu.VMEM((K, 80), jnp.float32),
        pltpu.VMEM((K, 16), jnp.float32),
        pltpu.VMEM((K, 80), jnp.float32),
        pltpu.VMEM_SHARED((NACC, 80), jnp.float32),
        pltpu.SemaphoreType.DMA,
        pltpu.SemaphoreType.DMA,
        pltpu.SemaphoreType.DMA,
        pltpu.SemaphoreType.DMA,
    ])

_edge2 = pl.kernel(
    _edge2_body,
    out_type=(jax.ShapeDtypeStruct((NACC, 16), jnp.float32),
              jax.ShapeDtypeStruct((NACC, 16), jnp.float32)),
    mesh=_MESH,
    compiler_params=_SC_PARAMS,
    scratch_types=[
        pltpu.VMEM((NCHUNK, K), jnp.int32),
        pltpu.VMEM((NCHUNK, K), jnp.int32),
        pltpu.VMEM((K, 16), jnp.float32),
        pltpu.VMEM((K, 16), jnp.float32),
        pltpu.VMEM((K, 16), jnp.float32),
        pltpu.VMEM((K, 16), jnp.float32),
        pltpu.VMEM((K, 16), jnp.float32),
        pltpu.VMEM((K, 16), jnp.float32),
        pltpu.VMEM_SHARED((NACC, 16), jnp.float32),
        pltpu.SemaphoreType.DMA,
        pltpu.SemaphoreType.DMA,
        pltpu.SemaphoreType.DMA,
        pltpu.SemaphoreType.DMA,
    ])


@jax.jit
def kernel(x, edge_index, W1, att_src1, att_dst1, b1,
           W2, att_src2, att_dst2, b2, Wl, bl):
    f32 = jnp.float32
    ei = edge_index.astype(jnp.int32)
    src = ei[0].reshape(NW, NCHUNK, K)
    dst = ei[1].reshape(NW, NCHUNK, K)
    x_pad = jnp.pad(x, ((0, NP - N), (0, 0)))

    H1, C1 = att_src1.shape[1], att_src1.shape[2]
    eyeH = jnp.eye(H1, dtype=f32)
    As = (eyeH[:, None, :] * att_src1[0][:, :, None]).reshape(H1 * C1, H1)
    Ad = (eyeH[:, None, :] * att_dst1[0][:, :, None]).reshape(H1 * C1, H1)
    e8 = jnp.kron(jnp.eye(8, dtype=f32), jnp.ones((1, 8), f32))
    b1m = jnp.broadcast_to(b1[None, :], (8, 64))
    w2pad = jnp.pad(W2, ((0, 0), (0, 8 - W2.shape[1])))
    p2 = jnp.zeros((8, 16), f32)
    p2 = p2.at[jnp.arange(3), jnp.arange(3)].set(1.0)
    p2 = p2.at[0:3, 3].set(att_src2[0, 0, :])
    p2 = p2.at[0:3, 4].set(att_dst2[0, 0, :])
    sel3 = jnp.zeros((16, 16), f32).at[jnp.arange(1, 4),
                                       jnp.arange(0, 3)].set(1.0)
    sel03 = jnp.zeros((16, 16), f32).at[jnp.arange(0, 3),
                                        jnp.arange(0, 3)].set(1.0)
    b2e = jnp.zeros((8, 16), f32).at[:, 0:3].set(
        jnp.broadcast_to(b2[None, :], (8, 3)))
    wl16 = jnp.zeros((16, 8), f32).at[0:3, :].set(
        jnp.broadcast_to(Wl, (3, 8)))
    c0 = jnp.broadcast_to(bl.reshape(1, 1), (8, 8))

    tab1, ad1 = pl.pallas_call(
        _node1_body,
        grid=(GRID,),
        in_specs=[
            pl.BlockSpec((BLK, 128), lambda i: (i, 0)),
            pl.BlockSpec((128, 64), lambda i: (0, 0)),
            pl.BlockSpec((64, 8), lambda i: (0, 0)),
            pl.BlockSpec((64, 8), lambda i: (0, 0)),
        ],
        out_specs=[
            pl.BlockSpec((BLK, 80), lambda i: (i, 0)),
            pl.BlockSpec((BLK, 16), lambda i: (i, 0)),
        ],
        out_shape=[
            jax.ShapeDtypeStruct((NP, 80), f32),
            jax.ShapeDtypeStruct((NP, 16), f32),
        ],
    )(x_pad, W1, As, Ad)

    acc1a, acc1b = _edge1(src, dst, tab1, ad1)

    tab2 = pl.pallas_call(
        _norm1_body,
        grid=(GRID,),
        in_specs=[
            pl.BlockSpec((BLK, 80), lambda i: (i, 0)),
            pl.BlockSpec((BLK, 80), lambda i: (i, 0)),
            pl.BlockSpec((BLK, 80), lambda i: (i, 0)),
            pl.BlockSpec((BLK, 16), lambda i: (i, 0)),
            pl.BlockSpec((8, 64), lambda i: (0, 0)),
            pl.BlockSpec((8, 64), lambda i: (0, 0)),
            pl.BlockSpec((64, 8), lambda i: (0, 0)),
            pl.BlockSpec((8, 16), lambda i: (0, 0)),
        ],
        out_specs=pl.BlockSpec((BLK, 16), lambda i: (i, 0)),
        out_shape=jax.ShapeDtypeStruct((NP, 16), f32),
    )(acc1a, acc1b, tab1, ad1, e8, b1m, w2pad, p2)

    acc2a, acc2b = _edge2(src, dst, tab2)

    out = pl.pallas_call(
        _final_body,
        grid=(GRID,),
        in_specs=[
            pl.BlockSpec((BLK, 16), lambda i: (i, 0)),
            pl.BlockSpec((BLK, 16), lambda i: (i, 0)),
            pl.BlockSpec((BLK, 16), lambda i: (i, 0)),
            pl.BlockSpec((16, 16), lambda i: (0, 0)),
            pl.BlockSpec((16, 16), lambda i: (0, 0)),
            pl.BlockSpec((8, 16), lambda i: (0, 0)),
            pl.BlockSpec((16, 8), lambda i: (0, 0)),
            pl.BlockSpec((8, 8), lambda i: (0, 0)),
        ],
        out_specs=pl.BlockSpec((BLK, 8), lambda i: (i, 0)),
        out_shape=jax.ShapeDtypeStruct((NP, 8), f32),
    )(acc2a, acc2b, tab2, sel3, sel03, b2e, wl16, c0)

    return out[:N, 0:1]
